# baseline (speedup 1.0000x reference)
import functools

import jax
import jax.numpy as jnp
from jax import lax
from jax.experimental import pallas as pl
from jax.experimental.pallas import tpu as pltpu
from jax.experimental.pallas import tpu_sc as plsc

_N = 10000
_NP = 10240
_E = 320000
_FI = 128
_HID = 32
_NC = 40
_DP = 48
_CH = 125
_KW = _E // 32 // _CH
_RPT = _NP // 16
_NBUF = 8
_NGRP = _KW // _NBUF

_mesh = plsc.VectorSubcoreMesh(core_axis_name="c", subcore_axis_name="s")
_sc_params = pltpu.CompilerParams(use_tc_tiling_on_sc=False)


@functools.partial(
    pl.kernel,
    mesh=_mesh,
    compiler_params=_sc_params,
    out_type=jax.ShapeDtypeStruct((2 * _NP,), jnp.float32),
    scratch_types=[
        pltpu.VMEM((_KW, _CH), jnp.int32),
        pltpu.VMEM((128,), jnp.float32),
        pltpu.VMEM((_RPT,), jnp.float32),
        pltpu.VMEM_SHARED((_NP,), jnp.float32),
    ] + [pltpu.SemaphoreType.DMA] * _NBUF,
)
def _deg_sc(e_hbm, out_hbm, idx_v, ones_v, buf_v, deg_sh, *ssem):
    c = lax.axis_index("c")
    s = lax.axis_index("s")
    w = c * 16 + s
    ones16 = jnp.ones((16,), jnp.float32)
    zero16 = jnp.zeros((16,), jnp.float32)
    for k in range(8):
        ones_v[pl.ds(k * 16, 16)] = ones16
    ones_c = ones_v.at[pl.ds(0, _CH)]

    def zb(i, _):
        buf_v[pl.ds(i * 16, 16)] = zero16
        return 0

    lax.fori_loop(0, _RPT // 16, zb, 0)
    pltpu.sync_copy(buf_v, deg_sh.at[pl.ds(s * _RPT, _RPT)])
    pltpu.sync_copy(e_hbm.at[1, w], idx_v)
    plsc.subcore_barrier()

    def grp(i, _):
        for b in range(_NBUF):
            j = i * _NBUF + b

            @pl.when(i > 0)
            def _():
                pltpu.make_async_copy(
                    ones_c, deg_sh.at[idx_v.at[j]], ssem[b]).wait()

            pltpu.async_copy(ones_c, deg_sh.at[idx_v.at[j]], ssem[b],
                             add=True)
        return 0

    lax.fori_loop(0, _NGRP, grp, 0)
    for b in range(_NBUF):
        pltpu.make_async_copy(ones_c, deg_sh.at[idx_v.at[b]], ssem[b]).wait()
    plsc.subcore_barrier()
    pltpu.sync_copy(deg_sh.at[pl.ds(s * _RPT, _RPT)], buf_v)
    pltpu.sync_copy(buf_v, out_hbm.at[pl.ds(c * _NP + s * _RPT, _RPT)])


def _make_agg(D):
    @functools.partial(
        pl.kernel,
        mesh=_mesh,
        compiler_params=_sc_params,
        out_type=jax.ShapeDtypeStruct((2 * _NP, D), jnp.float32),
        scratch_types=[
            pltpu.VMEM((_KW, _CH), jnp.int32),
            pltpu.VMEM((_KW, _CH), jnp.int32),
            pltpu.VMEM((_NBUF, _CH, D), jnp.float32),
            pltpu.VMEM((_RPT, D), jnp.float32),
            pltpu.VMEM_SHARED((_NP, D), jnp.float32),
        ] + [pltpu.SemaphoreType.DMA] * (2 * _NBUF),
    )
    def agg_k(e_hbm, m_hbm, out_hbm, si_v, di_v, rows_v, buf_v,
              agg_sh, *sems):
        gsem = sems[:_NBUF]
        ssem = sems[_NBUF:]
        c = lax.axis_index("c")
        s = lax.axis_index("s")
        w = c * 16 + s
        zero16 = jnp.zeros((16,), jnp.float32)

        def zb(i, _):
            for k in range(D // 16):
                buf_v[i, pl.ds(k * 16, 16)] = zero16
            return 0

        lax.fori_loop(0, _RPT, zb, 0)
        pltpu.sync_copy(buf_v, agg_sh.at[pl.ds(s * _RPT, _RPT)])
        pltpu.sync_copy(e_hbm.at[0, w], si_v)
        pltpu.sync_copy(e_hbm.at[1, w], di_v)
        plsc.subcore_barrier()

        def _wait_gather(j, b):
            pltpu.make_async_copy(
                m_hbm.at[si_v.at[j]], rows_v.at[b], gsem[b]).wait()

        def _wait_scatter(j, b):
            pltpu.make_async_copy(
                rows_v.at[b], agg_sh.at[di_v.at[j]], ssem[b]).wait()

        for b in range(_NBUF):
            pltpu.async_copy(m_hbm.at[si_v.at[b]], rows_v.at[b], gsem[b])

        def grp(i, _):
            for b in range(_NBUF):
                j = i * _NBUF + b
                _wait_gather(j, b)
                pltpu.async_copy(rows_v.at[b], agg_sh.at[di_v.at[j]],
                                 ssem[b], add=True)
            for b in range(_NBUF):
                j = i * _NBUF + b
                jn = j + _NBUF
                _wait_scatter(j, b)
                pltpu.async_copy(m_hbm.at[si_v.at[jn]], rows_v.at[b], gsem[b])
            return 0

        lax.fori_loop(0, _NGRP - 1, grp, 0)
        for b in range(_NBUF):
            j = (_NGRP - 1) * _NBUF + b
            _wait_gather(j, b)
            pltpu.async_copy(rows_v.at[b], agg_sh.at[di_v.at[j]],
                             ssem[b], add=True)
        for b in range(_NBUF):
            j = (_NGRP - 1) * _NBUF + b
            _wait_scatter(j, b)
        plsc.subcore_barrier()
        pltpu.sync_copy(agg_sh.at[pl.ds(s * _RPT, _RPT)], buf_v)
        pltpu.sync_copy(buf_v, out_hbm.at[pl.ds(c * _NP + s * _RPT, _RPT)])

    return agg_k


_agg32 = _make_agg(_HID)
_agg48 = _make_agg(_DP)


def _tc0_body(x_ref, w_ref, h_ref):
    h_ref[...] = jnp.dot(x_ref[...], w_ref[...],
                         preferred_element_type=jnp.float32)


def _tc0(x, w1):
    return pl.pallas_call(
        _tc0_body,
        out_shape=jax.ShapeDtypeStruct((_N, _HID), jnp.float32),
    )(x, w1)


def _tc1_body(deg_ref, h_ref, m_ref, dinv_ref):
    deg = deg_ref[pl.ds(0, _NP)] + deg_ref[pl.ds(_NP, _NP)] + 1.0
    dinv = lax.rsqrt(deg)
    m_ref[pl.ds(0, _N), :] = h_ref[...] * dinv[:_N, None]
    m_ref[pl.ds(_N, _NP - _N), :] = jnp.zeros((_NP - _N, _HID), jnp.float32)
    dinv_ref[...] = dinv


def _tc1(degp, h1):
    return pl.pallas_call(
        _tc1_body,
        out_shape=[
            jax.ShapeDtypeStruct((_NP, _HID), jnp.float32),
            jax.ShapeDtypeStruct((_NP,), jnp.float32),
        ],
    )(degp, h1)


def _tc2_body(p_ref, m_ref, dinv_ref, b1_ref, w2_ref, out_ref):
    agg = p_ref[pl.ds(0, _NP), :] + p_ref[pl.ds(_NP, _NP), :] + m_ref[...]
    dinv = dinv_ref[...]
    o1 = agg * dinv[:, None] + b1_ref[...][None, :]
    h = jax.nn.sigmoid(o1)
    w2p = jnp.concatenate(
        [w2_ref[...], jnp.zeros((_HID, _DP - _NC), jnp.float32)], axis=1)
    h2 = jnp.dot(h, w2p, preferred_element_type=jnp.float32)
    out_ref[...] = h2 * dinv[:, None]


def _tc2(p1, m1, dinv, b1, w2):
    return pl.pallas_call(
        _tc2_body,
        out_shape=jax.ShapeDtypeStruct((_NP, _DP), jnp.float32),
    )(p1, m1, dinv, b1, w2)


def _tc3_body(p_ref, m_ref, dinv_ref, b_ref, out_ref):
    agg = (p_ref[pl.ds(0, _N), :] + p_ref[pl.ds(_NP, _N), :]
           + m_ref[pl.ds(0, _N), :])
    o = agg * dinv_ref[pl.ds(0, _N)][:, None]
    out_ref[...] = o[:, : _NC] + b_ref[...][None, :]


def _tc3(p2, m2, dinv, b2):
    return pl.pallas_call(
        _tc3_body,
        out_shape=jax.ShapeDtypeStruct((_N, _NC), jnp.float32),
    )(p2, m2, dinv, b2)


def kernel(x, edge_index, W1, b1, W2, b2):
    e3d = edge_index.reshape(2, 32, _KW, _CH)
    degp = _deg_sc(e3d)
    h1 = _tc0(x, W1)
    m1, dinv = _tc1(degp, h1)
    p1 = _agg32(e3d, m1)
    m2 = _tc2(p1, m1, dinv, b1, W2)
    p2 = _agg48(e3d, m2)
    return _tc3(p2, m2, dinv, b2)


# speedup vs baseline: 1.0073x; 1.0073x over previous
import functools

import jax
import jax.numpy as jnp
from jax import lax
from jax.experimental import pallas as pl
from jax.experimental.pallas import tpu as pltpu
from jax.experimental.pallas import tpu_sc as plsc

_N = 10000
_NP = 10240
_E = 320000
_FI = 128
_HID = 32
_NC = 40
_DP = 48
_CH = 125
_KW = _E // 32 // _CH
_RPT = _NP // 16
_NBUF = 8
_NGRP = _KW // _NBUF

_mesh = plsc.VectorSubcoreMesh(core_axis_name="c", subcore_axis_name="s")
_sc_params = pltpu.CompilerParams(use_tc_tiling_on_sc=False)


@functools.partial(
    pl.kernel,
    mesh=_mesh,
    compiler_params=_sc_params,
    out_type=jax.ShapeDtypeStruct((2 * _NP,), jnp.float32),
    scratch_types=[
        pltpu.VMEM((_KW, _CH), jnp.int32),
        pltpu.VMEM((128,), jnp.float32),
        pltpu.VMEM((_RPT,), jnp.float32),
        pltpu.VMEM_SHARED((_NP,), jnp.float32),
    ] + [pltpu.SemaphoreType.DMA] * _NBUF,
)
def _deg_sc(e_hbm, out_hbm, idx_v, ones_v, buf_v, deg_sh, *ssem):
    c = lax.axis_index("c")
    s = lax.axis_index("s")
    w = c * 16 + s
    ones16 = jnp.ones((16,), jnp.float32)
    zero16 = jnp.zeros((16,), jnp.float32)
    for k in range(8):
        ones_v[pl.ds(k * 16, 16)] = ones16
    ones_c = ones_v.at[pl.ds(0, _CH)]

    def zb(i, _):
        buf_v[pl.ds(i * 16, 16)] = zero16
        return 0

    lax.fori_loop(0, _RPT // 16, zb, 0)
    pltpu.sync_copy(buf_v, deg_sh.at[pl.ds(s * _RPT, _RPT)])
    pltpu.sync_copy(e_hbm.at[1, w], idx_v)
    plsc.subcore_barrier()

    def grp(i, _):
        for b in range(_NBUF):
            j = i * _NBUF + b

            @pl.when(i > 0)
            def _():
                pltpu.make_async_copy(
                    ones_c, deg_sh.at[idx_v.at[j]], ssem[b]).wait()

            pltpu.async_copy(ones_c, deg_sh.at[idx_v.at[j]], ssem[b],
                             add=True)
        return 0

    lax.fori_loop(0, _NGRP, grp, 0)
    for b in range(_NBUF):
        pltpu.make_async_copy(ones_c, deg_sh.at[idx_v.at[b]], ssem[b]).wait()
    plsc.subcore_barrier()
    pltpu.sync_copy(deg_sh.at[pl.ds(s * _RPT, _RPT)], buf_v)
    pltpu.sync_copy(buf_v, out_hbm.at[pl.ds(c * _NP + s * _RPT, _RPT)])


def _make_agg(D):
    @functools.partial(
        pl.kernel,
        mesh=_mesh,
        compiler_params=_sc_params,
        out_type=jax.ShapeDtypeStruct((2 * _NP, D), jnp.float32),
        scratch_types=[
            pltpu.VMEM((_KW, _CH), jnp.int32),
            pltpu.VMEM((_KW, _CH), jnp.int32),
            pltpu.VMEM((_NBUF, _CH, D), jnp.float32),
            pltpu.VMEM((_RPT, D), jnp.float32),
            pltpu.VMEM_SHARED((_NP, D), jnp.float32),
        ] + [pltpu.SemaphoreType.DMA] * (2 * _NBUF),
    )
    def agg_k(e_hbm, m_hbm, out_hbm, si_v, di_v, rows_v, buf_v,
              agg_sh, *sems):
        gsem = sems[:_NBUF]
        ssem = sems[_NBUF:]
        c = lax.axis_index("c")
        s = lax.axis_index("s")
        w = c * 16 + s
        zero16 = jnp.zeros((16,), jnp.float32)

        def zb(i, _):
            for k in range(D // 16):
                buf_v[i, pl.ds(k * 16, 16)] = zero16
            return 0

        lax.fori_loop(0, _RPT, zb, 0)
        pltpu.sync_copy(buf_v, agg_sh.at[pl.ds(s * _RPT, _RPT)])
        pltpu.sync_copy(e_hbm.at[0, w], si_v)
        pltpu.sync_copy(e_hbm.at[1, w], di_v)
        plsc.subcore_barrier()

        def _wait_gather(j, b):
            pltpu.make_async_copy(
                m_hbm.at[si_v.at[j]], rows_v.at[b], gsem[b]).wait()

        def _wait_scatter(j, b):
            pltpu.make_async_copy(
                rows_v.at[b], agg_sh.at[di_v.at[j]], ssem[b]).wait()

        for b in range(_NBUF):
            pltpu.async_copy(m_hbm.at[si_v.at[b]], rows_v.at[b], gsem[b])

        def grp(i, _):
            for b in range(_NBUF):
                j = i * _NBUF + b
                _wait_gather(j, b)
                pltpu.async_copy(rows_v.at[b], agg_sh.at[di_v.at[j]],
                                 ssem[b], add=True)
            for b in range(_NBUF):
                j = i * _NBUF + b
                jn = j + _NBUF
                _wait_scatter(j, b)
                pltpu.async_copy(m_hbm.at[si_v.at[jn]], rows_v.at[b], gsem[b])
            return 0

        lax.fori_loop(0, _NGRP - 1, grp, 0)
        for b in range(_NBUF):
            j = (_NGRP - 1) * _NBUF + b
            _wait_gather(j, b)
            pltpu.async_copy(rows_v.at[b], agg_sh.at[di_v.at[j]],
                             ssem[b], add=True)
        for b in range(_NBUF):
            j = (_NGRP - 1) * _NBUF + b
            _wait_scatter(j, b)
        plsc.subcore_barrier()
        pltpu.sync_copy(agg_sh.at[pl.ds(s * _RPT, _RPT)], buf_v)
        pltpu.sync_copy(buf_v, out_hbm.at[pl.ds(c * _NP + s * _RPT, _RPT)])

    return agg_k


_agg32 = _make_agg(_HID)
_agg48 = _make_agg(_DP)


def _tc1_body(deg_ref, x_ref, w_ref, m_ref, dinv_ref):
    deg = deg_ref[pl.ds(0, _NP)] + deg_ref[pl.ds(_NP, _NP)] + 1.0
    dinv = lax.rsqrt(deg)
    h = jnp.dot(x_ref[...], w_ref[...], preferred_element_type=jnp.float32)
    m_ref[pl.ds(0, _N), :] = h * dinv[:_N, None]
    m_ref[pl.ds(_N, _NP - _N), :] = jnp.zeros((_NP - _N, _HID), jnp.float32)
    dinv_ref[...] = dinv


def _tc1(degp, x, w1):
    return pl.pallas_call(
        _tc1_body,
        out_shape=[
            jax.ShapeDtypeStruct((_NP, _HID), jnp.float32),
            jax.ShapeDtypeStruct((_NP,), jnp.float32),
        ],
    )(degp, x, w1)


def _tc2_body(p_ref, m_ref, dinv_ref, b1_ref, w2_ref, out_ref):
    agg = p_ref[pl.ds(0, _NP), :] + p_ref[pl.ds(_NP, _NP), :] + m_ref[...]
    dinv = dinv_ref[...]
    o1 = agg * dinv[:, None] + b1_ref[...][None, :]
    h = jax.nn.sigmoid(o1)
    w2p = jnp.concatenate(
        [w2_ref[...], jnp.zeros((_HID, _DP - _NC), jnp.float32)], axis=1)
    h2 = jnp.dot(h, w2p, preferred_element_type=jnp.float32)
    out_ref[...] = h2 * dinv[:, None]


def _tc2(p1, m1, dinv, b1, w2):
    return pl.pallas_call(
        _tc2_body,
        out_shape=jax.ShapeDtypeStruct((_NP, _DP), jnp.float32),
    )(p1, m1, dinv, b1, w2)


def _tc3_body(p_ref, m_ref, dinv_ref, b_ref, out_ref):
    agg = (p_ref[pl.ds(0, _N), :] + p_ref[pl.ds(_NP, _N), :]
           + m_ref[pl.ds(0, _N), :])
    o = agg * dinv_ref[pl.ds(0, _N)][:, None]
    out_ref[...] = o[:, : _NC] + b_ref[...][None, :]


def _tc3(p2, m2, dinv, b2):
    return pl.pallas_call(
        _tc3_body,
        out_shape=jax.ShapeDtypeStruct((_N, _NC), jnp.float32),
    )(p2, m2, dinv, b2)


def kernel(x, edge_index, W1, b1, W2, b2):
    e3d = edge_index.reshape(2, 32, _KW, _CH)
    degp = _deg_sc(e3d)
    m1, dinv = _tc1(degp, x, W1)
    p1 = _agg32(e3d, m1)
    m2 = _tc2(p1, m1, dinv, b1, W2)
    p2 = _agg48(e3d, m2)
    return _tc3(p2, m2, dinv, b2)
